# baseline (speedup 1.0000x reference)
import functools

import jax
import jax.numpy as jnp
from jax import lax
from jax.experimental import pallas as pl
from jax.experimental.pallas import tpu as pltpu
from jax.experimental.pallas import tpu_sc as plsc

NC = 2
NS = 16
NW = NC * NS
LN = 32
CH_A = 175
CH_B = 455


def _sc_mesh():
    return plsc.VectorSubcoreMesh(core_axis_name="c", subcore_axis_name="s")


def _sc_degree(dst3, zeros_blk, ones_blk):
    _, CH, _ = dst3.shape
    NP = zeros_blk.shape[0] * NS
    PT = NP // NS

    @functools.partial(
        pl.kernel,
        mesh=_sc_mesh(),
        out_type=jax.ShapeDtypeStruct((NC, NP, 8), jnp.float32),
        compiler_params=pltpu.CompilerParams(use_tc_tiling_on_sc=False),
        scratch_types=[
            pltpu.VMEM((CH, LN), jnp.int32),
            pltpu.VMEM((LN, 8), jnp.float32),
            pltpu.VMEM_SHARED((NP, 8), jnp.float32),
        ],
    )
    def deg_kernel(dst_h, zb_h, ones_h, out_h, dst_v, ones_v, acc):
        cid = lax.axis_index("c")
        sid = lax.axis_index("s")
        wid = cid * NS + sid
        chc = jnp.where(cid == 0, CH_A, CH_B)
        pltpu.sync_copy(zb_h, acc.at[pl.ds(sid * PT, PT)])
        pltpu.sync_copy(dst_h.at[wid], dst_v)
        pltpu.sync_copy(ones_h, ones_v)
        plsc.subcore_barrier()

        def step(j, carry):
            for b in range(5):
                c = j * 5 + b
                pltpu.sync_copy(ones_v, acc.at[dst_v.at[c]], add=True)
            return carry

        lax.fori_loop(0, chc // 5, step, 0)
        plsc.subcore_barrier()
        pltpu.sync_copy(acc.at[pl.ds(sid * PT, PT)],
                        out_h.at[cid, pl.ds(sid * PT, PT)])

    return deg_kernel(dst3, zeros_blk, ones_blk)


def _sc_propagate(y, src3, dst3, zeros_blk, NBUF):
    NP, F = y.shape
    _, CH, _ = src3.shape
    PT = NP // NS

    @functools.partial(
        pl.kernel,
        mesh=_sc_mesh(),
        out_type=jax.ShapeDtypeStruct((NC, NP, F), jnp.float32),
        compiler_params=pltpu.CompilerParams(use_tc_tiling_on_sc=False),
        scratch_types=(
            [pltpu.VMEM((CH, LN), jnp.int32),
             pltpu.VMEM((CH, LN), jnp.int32)]
            + [pltpu.VMEM((LN, F), jnp.float32) for _ in range(NBUF)]
            + [pltpu.SemaphoreType.DMA for _ in range(NBUF)]
            + [pltpu.VMEM_SHARED((NP, F), jnp.float32)]
        ),
    )
    def prop_kernel(y_h, src_h, dst_h, zb_h, out_h, src_v, dst_v, *rest):
        rows = rest[:NBUF]
        sems = rest[NBUF:2 * NBUF]
        acc = rest[2 * NBUF]
        cid = lax.axis_index("c")
        sid = lax.axis_index("s")
        wid = cid * NS + sid
        chc = jnp.where(cid == 0, CH_A, CH_B)
        pltpu.sync_copy(zb_h, acc.at[pl.ds(sid * PT, PT)])
        pltpu.sync_copy(src_h.at[wid], src_v)
        pltpu.sync_copy(dst_h.at[wid], dst_v)
        plsc.subcore_barrier()

        for b in range(NBUF - 1):
            pltpu.async_copy(y_h.at[src_v.at[b]], rows[b], sems[b])

        def step(j, carry):
            for b in range(NBUF):
                c = j * NBUF + b
                pltpu.make_async_copy(y_h.at[src_v.at[c]],
                                      rows[b], sems[b]).wait()
                cn = jnp.minimum(c + NBUF - 1, chc - 1)
                bn = (b + NBUF - 1) % NBUF
                pltpu.async_copy(y_h.at[src_v.at[cn]], rows[bn], sems[bn])
                pltpu.sync_copy(rows[b], acc.at[dst_v.at[c]], add=True)
            return carry

        lax.fori_loop(0, chc // NBUF, step, 0)
        for k in range(NBUF - 1):
            b = k % NBUF
            pltpu.make_async_copy(y_h.at[src_v.at[0]],
                                  rows[b], sems[b]).wait()
        plsc.subcore_barrier()
        pltpu.sync_copy(acc.at[pl.ds(sid * PT, PT)],
                        out_h.at[cid, pl.ds(sid * PT, PT)])

    return prop_kernel(y, src3, dst3, zeros_blk)


def _edge_layout(edge_index, N):
    E = edge_index.shape[1]
    cap_a = NS * CH_A * LN
    cap = cap_a + NS * CH_B * LN
    src = jnp.pad(edge_index[0], (0, cap - E))
    dst = jnp.pad(edge_index[1], (0, cap - E), constant_values=N)

    CHM = max(CH_A, CH_B)

    def shape3(a):
        a0 = a[:cap_a].reshape(NS, CH_A, LN)
        a1 = a[cap_a:].reshape(NS, CH_B, LN)
        a0 = jnp.pad(a0, ((0, 0), (0, CHM - CH_A), (0, 0)))
        a1 = jnp.pad(a1, ((0, 0), (0, CHM - CH_B), (0, 0)))
        return jnp.concatenate([a0, a1], axis=0)

    return shape3(src), shape3(dst)


def _deg_inv_sqrt(p0_r, p1_r):
    deg = p0_r[:, 0:1] + p1_r[:, 0:1] + 1.0
    return lax.rsqrt(jnp.maximum(deg, 1e-12))


def _tc_first(xp, W, p0, p1, blk_r):
    NP, K = xp.shape
    F = W.shape[1]
    nb = NP // blk_r

    def body(x_r, w_r, p0_r, p1_r, o_r):
        d = _deg_inv_sqrt(p0_r, p1_r)
        o_r[...] = jnp.dot(x_r[...], w_r[...],
                           preferred_element_type=jnp.float32) * d

    return pl.pallas_call(
        body,
        grid=(nb,),
        in_specs=[
            pl.BlockSpec((blk_r, K), lambda i: (i, 0)),
            pl.BlockSpec((K, F), lambda i: (0, 0)),
            pl.BlockSpec((blk_r, 8), lambda i: (i, 0)),
            pl.BlockSpec((blk_r, 8), lambda i: (i, 0)),
        ],
        out_specs=pl.BlockSpec((blk_r, F), lambda i: (i, 0)),
        out_shape=jax.ShapeDtypeStruct((NP, F), jnp.float32),
    )(xp, W, p0, p1)


def _tc_mid(s0, s1, yp, p0, p1, b2d, W, blk_r):
    NP, K = yp.shape
    F = W.shape[1]
    nb = NP // blk_r

    def body(s0_r, s1_r, y_r, p0_r, p1_r, b_r, w_r, o_r):
        d = _deg_inv_sqrt(p0_r, p1_r)
        z = d * (s0_r[...] + s1_r[...] + y_r[...]) + b_r[...]
        a = jnp.maximum(z, 0.0)
        o_r[...] = jnp.dot(a, w_r[...],
                           preferred_element_type=jnp.float32) * d

    return pl.pallas_call(
        body,
        grid=(nb,),
        in_specs=[
            pl.BlockSpec((blk_r, K), lambda i: (i, 0)),
            pl.BlockSpec((blk_r, K), lambda i: (i, 0)),
            pl.BlockSpec((blk_r, K), lambda i: (i, 0)),
            pl.BlockSpec((blk_r, 8), lambda i: (i, 0)),
            pl.BlockSpec((blk_r, 8), lambda i: (i, 0)),
            pl.BlockSpec((1, K), lambda i: (0, 0)),
            pl.BlockSpec((K, F), lambda i: (0, 0)),
        ],
        out_specs=pl.BlockSpec((blk_r, F), lambda i: (i, 0)),
        out_shape=jax.ShapeDtypeStruct((NP, F), jnp.float32),
    )(s0, s1, yp, p0, p1, b2d, W)


def _tc_final(s0, s1, yp, p0, p1, b2d, blk_r):
    NP, K = yp.shape
    nb = NP // blk_r

    def body(s0_r, s1_r, y_r, p0_r, p1_r, b_r, o_r):
        d = _deg_inv_sqrt(p0_r, p1_r)
        z = d * (s0_r[...] + s1_r[...] + y_r[...]) + b_r[...]
        m = jnp.max(z, axis=1, keepdims=True)
        e = jnp.exp(z - m)
        lse = jnp.log(jnp.sum(e, axis=1, keepdims=True)) + m
        o_r[...] = z - lse

    return pl.pallas_call(
        body,
        grid=(nb,),
        in_specs=[
            pl.BlockSpec((blk_r, K), lambda i: (i, 0)),
            pl.BlockSpec((blk_r, K), lambda i: (i, 0)),
            pl.BlockSpec((blk_r, K), lambda i: (i, 0)),
            pl.BlockSpec((blk_r, 8), lambda i: (i, 0)),
            pl.BlockSpec((blk_r, 8), lambda i: (i, 0)),
            pl.BlockSpec((1, K), lambda i: (0, 0)),
        ],
        out_specs=pl.BlockSpec((blk_r, K), lambda i: (i, 0)),
        out_shape=jax.ShapeDtypeStruct((NP, K), jnp.float32),
    )(s0, s1, yp, p0, p1, b2d)


def kernel(x, edge_index, W1, b1, W2, b2, W3, b3):
    N, K = x.shape
    E = edge_index.shape[1]

    NP = ((N + NS * 8 - 1) // (NS * 8)) * (NS * 8)
    PT = NP // NS

    src3, dst3 = _edge_layout(edge_index, N)
    xp = jnp.pad(x, ((0, NP - N), (0, 0)))

    z8 = jnp.zeros((PT, 8), jnp.float32)
    ones8 = jnp.ones((LN, 8), jnp.float32)
    zK = jnp.zeros((PT, K), jnp.float32)
    F2 = W2.shape[1]
    zF = jnp.zeros((PT, F2), jnp.float32)

    blk_r = PT

    pdeg = _sc_degree(dst3, z8, ones8)
    p0, p1 = pdeg[0], pdeg[1]

    y1 = _tc_first(xp, W1, p0, p1, blk_r)
    s1 = _sc_propagate(y1, src3, dst3, zK, 5)
    y2 = _tc_mid(s1[0], s1[1], y1, p0, p1, b1.reshape(1, -1), W2, blk_r)
    s2 = _sc_propagate(y2, src3, dst3, zF, 7)
    y3 = _tc_mid(s2[0], s2[1], y2, p0, p1, b2.reshape(1, -1), W3, blk_r)
    s3 = _sc_propagate(y3, src3, dst3, zF, 7)
    out = _tc_final(s3[0], s3[1], y3, p0, p1, b3.reshape(1, -1), blk_r)
    return out[:N]


# speedup vs baseline: 1.1757x; 1.1757x over previous
import functools

import jax
import jax.numpy as jnp
from jax import lax
from jax.experimental import pallas as pl
from jax.experimental.pallas import tpu as pltpu
from jax.experimental.pallas import tpu_sc as plsc

NC = 2
NS = 16
NW = NC * NS
LN = 32
CH_A = 455
CH_B = 175


def _sc_mesh():
    return plsc.VectorSubcoreMesh(core_axis_name="c", subcore_axis_name="s")


def _sc_degree(dst3, zeros_blk, ones_blk):
    _, CH, _ = dst3.shape
    NP = zeros_blk.shape[0] * NS
    PT = NP // NS

    @functools.partial(
        pl.kernel,
        mesh=_sc_mesh(),
        out_type=jax.ShapeDtypeStruct((NC, NP, 8), jnp.float32),
        compiler_params=pltpu.CompilerParams(use_tc_tiling_on_sc=False),
        scratch_types=[
            pltpu.VMEM((CH, LN), jnp.int32),
            pltpu.VMEM((LN, 8), jnp.float32),
            pltpu.VMEM_SHARED((NP, 8), jnp.float32),
        ],
    )
    def deg_kernel(dst_h, zb_h, ones_h, out_h, dst_v, ones_v, acc):
        cid = lax.axis_index("c")
        sid = lax.axis_index("s")
        wid = cid * NS + sid
        chc = jnp.where(cid == 0, CH_A, CH_B)
        pltpu.sync_copy(zb_h, acc.at[pl.ds(sid * PT, PT)])
        pltpu.sync_copy(dst_h.at[wid], dst_v)
        pltpu.sync_copy(ones_h, ones_v)
        plsc.subcore_barrier()

        def step(j, carry):
            for b in range(5):
                c = j * 5 + b
                pltpu.sync_copy(ones_v, acc.at[dst_v.at[c]], add=True)
            return carry

        lax.fori_loop(0, chc // 5, step, 0)
        plsc.subcore_barrier()
        pltpu.sync_copy(acc.at[pl.ds(sid * PT, PT)],
                        out_h.at[cid, pl.ds(sid * PT, PT)])

    return deg_kernel(dst3, zeros_blk, ones_blk)


def _sc_propagate(y, src3, dst3, zeros_blk, NBUF):
    NP, F = y.shape
    _, CH, _ = src3.shape
    PT = NP // NS

    @functools.partial(
        pl.kernel,
        mesh=_sc_mesh(),
        out_type=jax.ShapeDtypeStruct((NC, NP, F), jnp.float32),
        compiler_params=pltpu.CompilerParams(use_tc_tiling_on_sc=False),
        scratch_types=(
            [pltpu.VMEM((CH, LN), jnp.int32),
             pltpu.VMEM((CH, LN), jnp.int32)]
            + [pltpu.VMEM((LN, F), jnp.float32) for _ in range(NBUF)]
            + [pltpu.SemaphoreType.DMA for _ in range(NBUF)]
            + [pltpu.VMEM_SHARED((NP, F), jnp.float32)]
        ),
    )
    def prop_kernel(y_h, src_h, dst_h, zb_h, out_h, src_v, dst_v, *rest):
        rows = rest[:NBUF]
        sems = rest[NBUF:2 * NBUF]
        acc = rest[2 * NBUF]
        cid = lax.axis_index("c")
        sid = lax.axis_index("s")
        wid = cid * NS + sid
        chc = jnp.where(cid == 0, CH_A, CH_B)
        pltpu.sync_copy(zb_h, acc.at[pl.ds(sid * PT, PT)])
        pltpu.sync_copy(src_h.at[wid], src_v)
        pltpu.sync_copy(dst_h.at[wid], dst_v)
        plsc.subcore_barrier()

        for b in range(NBUF - 1):
            pltpu.async_copy(y_h.at[src_v.at[b]], rows[b], sems[b])

        def step(j, carry):
            for b in range(NBUF):
                c = j * NBUF + b
                pltpu.make_async_copy(y_h.at[src_v.at[c]],
                                      rows[b], sems[b]).wait()
                cn = jnp.minimum(c + NBUF - 1, chc - 1)
                bn = (b + NBUF - 1) % NBUF
                pltpu.async_copy(y_h.at[src_v.at[cn]], rows[bn], sems[bn])
                pltpu.sync_copy(rows[b], acc.at[dst_v.at[c]], add=True)
            return carry

        lax.fori_loop(0, chc // NBUF, step, 0)
        for k in range(NBUF - 1):
            b = k % NBUF
            pltpu.make_async_copy(y_h.at[src_v.at[0]],
                                  rows[b], sems[b]).wait()
        plsc.subcore_barrier()
        pltpu.sync_copy(acc.at[pl.ds(sid * PT, PT)],
                        out_h.at[cid, pl.ds(sid * PT, PT)])

    return prop_kernel(y, src3, dst3, zeros_blk)


def _edge_layout(edge_index, N):
    E = edge_index.shape[1]
    cap_a = NS * CH_A * LN
    cap = cap_a + NS * CH_B * LN
    src = jnp.pad(edge_index[0], (0, cap - E))
    dst = jnp.pad(edge_index[1], (0, cap - E), constant_values=N)

    CHM = max(CH_A, CH_B)

    def shape3(a):
        a0 = a[:cap_a].reshape(NS, CH_A, LN)
        a1 = a[cap_a:].reshape(NS, CH_B, LN)
        a0 = jnp.pad(a0, ((0, 0), (0, CHM - CH_A), (0, 0)))
        a1 = jnp.pad(a1, ((0, 0), (0, CHM - CH_B), (0, 0)))
        return jnp.concatenate([a0, a1], axis=0)

    return shape3(src), shape3(dst)


def _deg_inv_sqrt(p0_r, p1_r):
    deg = p0_r[:, 0:1] + p1_r[:, 0:1] + 1.0
    return lax.rsqrt(jnp.maximum(deg, 1e-12))


def _tc_first(xp, W, p0, p1, blk_r):
    NP, K = xp.shape
    F = W.shape[1]
    nb = NP // blk_r

    def body(x_r, w_r, p0_r, p1_r, o_r):
        d = _deg_inv_sqrt(p0_r, p1_r)
        o_r[...] = jnp.dot(x_r[...], w_r[...],
                           preferred_element_type=jnp.float32) * d

    return pl.pallas_call(
        body,
        grid=(nb,),
        in_specs=[
            pl.BlockSpec((blk_r, K), lambda i: (i, 0)),
            pl.BlockSpec((K, F), lambda i: (0, 0)),
            pl.BlockSpec((blk_r, 8), lambda i: (i, 0)),
            pl.BlockSpec((blk_r, 8), lambda i: (i, 0)),
        ],
        out_specs=pl.BlockSpec((blk_r, F), lambda i: (i, 0)),
        out_shape=jax.ShapeDtypeStruct((NP, F), jnp.float32),
    )(xp, W, p0, p1)


def _tc_mid(s0, s1, yp, p0, p1, b2d, W, blk_r):
    NP, K = yp.shape
    F = W.shape[1]
    nb = NP // blk_r

    def body(s0_r, s1_r, y_r, p0_r, p1_r, b_r, w_r, o_r):
        d = _deg_inv_sqrt(p0_r, p1_r)
        z = d * (s0_r[...] + s1_r[...] + y_r[...]) + b_r[...]
        a = jnp.maximum(z, 0.0)
        o_r[...] = jnp.dot(a, w_r[...],
                           preferred_element_type=jnp.float32) * d

    return pl.pallas_call(
        body,
        grid=(nb,),
        in_specs=[
            pl.BlockSpec((blk_r, K), lambda i: (i, 0)),
            pl.BlockSpec((blk_r, K), lambda i: (i, 0)),
            pl.BlockSpec((blk_r, K), lambda i: (i, 0)),
            pl.BlockSpec((blk_r, 8), lambda i: (i, 0)),
            pl.BlockSpec((blk_r, 8), lambda i: (i, 0)),
            pl.BlockSpec((1, K), lambda i: (0, 0)),
            pl.BlockSpec((K, F), lambda i: (0, 0)),
        ],
        out_specs=pl.BlockSpec((blk_r, F), lambda i: (i, 0)),
        out_shape=jax.ShapeDtypeStruct((NP, F), jnp.float32),
    )(s0, s1, yp, p0, p1, b2d, W)


def _tc_final(s0, s1, yp, p0, p1, b2d, blk_r):
    NP, K = yp.shape
    nb = NP // blk_r

    def body(s0_r, s1_r, y_r, p0_r, p1_r, b_r, o_r):
        d = _deg_inv_sqrt(p0_r, p1_r)
        z = d * (s0_r[...] + s1_r[...] + y_r[...]) + b_r[...]
        m = jnp.max(z, axis=1, keepdims=True)
        e = jnp.exp(z - m)
        lse = jnp.log(jnp.sum(e, axis=1, keepdims=True)) + m
        o_r[...] = z - lse

    return pl.pallas_call(
        body,
        grid=(nb,),
        in_specs=[
            pl.BlockSpec((blk_r, K), lambda i: (i, 0)),
            pl.BlockSpec((blk_r, K), lambda i: (i, 0)),
            pl.BlockSpec((blk_r, K), lambda i: (i, 0)),
            pl.BlockSpec((blk_r, 8), lambda i: (i, 0)),
            pl.BlockSpec((blk_r, 8), lambda i: (i, 0)),
            pl.BlockSpec((1, K), lambda i: (0, 0)),
        ],
        out_specs=pl.BlockSpec((blk_r, K), lambda i: (i, 0)),
        out_shape=jax.ShapeDtypeStruct((NP, K), jnp.float32),
    )(s0, s1, yp, p0, p1, b2d)


def kernel(x, edge_index, W1, b1, W2, b2, W3, b3):
    N, K = x.shape
    E = edge_index.shape[1]

    NP = ((N + NS * 8 - 1) // (NS * 8)) * (NS * 8)
    PT = NP // NS

    src3, dst3 = _edge_layout(edge_index, N)
    xp = jnp.pad(x, ((0, NP - N), (0, 0)))

    z8 = jnp.zeros((PT, 8), jnp.float32)
    ones8 = jnp.ones((LN, 8), jnp.float32)
    zK = jnp.zeros((PT, K), jnp.float32)
    F2 = W2.shape[1]
    zF = jnp.zeros((PT, F2), jnp.float32)

    blk_r = PT

    pdeg = _sc_degree(dst3, z8, ones8)
    p0, p1 = pdeg[0], pdeg[1]

    y1 = _tc_first(xp, W1, p0, p1, blk_r)
    s1 = _sc_propagate(y1, src3, dst3, zK, 5)
    y2 = _tc_mid(s1[0], s1[1], y1, p0, p1, b1.reshape(1, -1), W2, blk_r)
    s2 = _sc_propagate(y2, src3, dst3, zF, 7)
    y3 = _tc_mid(s2[0], s2[1], y2, p0, p1, b2.reshape(1, -1), W3, blk_r)
    s3 = _sc_propagate(y3, src3, dst3, zF, 7)
    out = _tc_final(s3[0], s3[1], y3, p0, p1, b3.reshape(1, -1), blk_r)
    return out[:N]


# speedup vs baseline: 1.1779x; 1.0019x over previous
import functools

import jax
import jax.numpy as jnp
from jax import lax
from jax.experimental import pallas as pl
from jax.experimental.pallas import tpu as pltpu
from jax.experimental.pallas import tpu_sc as plsc

NC = 2
NS = 16
NW = NC * NS
LN = 32


def _sc_mesh():
    return plsc.VectorSubcoreMesh(core_axis_name="c", subcore_axis_name="s")


def _sc_degree(dst3, zeros_blk, ones_blk, CH_A, CH_B):
    _, CH, _ = dst3.shape
    NP = zeros_blk.shape[0] * NS
    PT = NP // NS

    @functools.partial(
        pl.kernel,
        mesh=_sc_mesh(),
        out_type=jax.ShapeDtypeStruct((NC, NP, 8), jnp.float32),
        compiler_params=pltpu.CompilerParams(use_tc_tiling_on_sc=False),
        scratch_types=[
            pltpu.VMEM((CH, LN), jnp.int32),
            pltpu.VMEM((LN, 8), jnp.float32),
            pltpu.VMEM_SHARED((NP, 8), jnp.float32),
        ],
    )
    def deg_kernel(dst_h, zb_h, ones_h, out_h, dst_v, ones_v, acc):
        cid = lax.axis_index("c")
        sid = lax.axis_index("s")
        wid = cid * NS + sid
        chc = jnp.where(cid == 0, CH_A, CH_B)
        pltpu.sync_copy(zb_h, acc.at[pl.ds(sid * PT, PT)])
        pltpu.sync_copy(dst_h.at[wid], dst_v)
        pltpu.sync_copy(ones_h, ones_v)
        plsc.subcore_barrier()

        def step(j, carry):
            for b in range(5):
                c = j * 5 + b
                pltpu.sync_copy(ones_v, acc.at[dst_v.at[c]], add=True)
            return carry

        lax.fori_loop(0, chc // 5, step, 0)
        plsc.subcore_barrier()
        pltpu.sync_copy(acc.at[pl.ds(sid * PT, PT)],
                        out_h.at[cid, pl.ds(sid * PT, PT)])

    return deg_kernel(dst3, zeros_blk, ones_blk)


def _sc_propagate(y, src3, dst3, zeros_blk, NBUF, CH_A, CH_B):
    NP, F = y.shape
    _, CH, _ = src3.shape
    PT = NP // NS

    @functools.partial(
        pl.kernel,
        mesh=_sc_mesh(),
        out_type=jax.ShapeDtypeStruct((NC, NP, F), jnp.float32),
        compiler_params=pltpu.CompilerParams(use_tc_tiling_on_sc=False),
        scratch_types=(
            [pltpu.VMEM((CH, LN), jnp.int32),
             pltpu.VMEM((CH, LN), jnp.int32)]
            + [pltpu.VMEM((LN, F), jnp.float32) for _ in range(NBUF)]
            + [pltpu.SemaphoreType.DMA for _ in range(NBUF)]
            + [pltpu.VMEM_SHARED((NP, F), jnp.float32)]
        ),
    )
    def prop_kernel(y_h, src_h, dst_h, zb_h, out_h, src_v, dst_v, *rest):
        rows = rest[:NBUF]
        sems = rest[NBUF:2 * NBUF]
        acc = rest[2 * NBUF]
        cid = lax.axis_index("c")
        sid = lax.axis_index("s")
        wid = cid * NS + sid
        chc = jnp.where(cid == 0, CH_A, CH_B)
        pltpu.sync_copy(zb_h, acc.at[pl.ds(sid * PT, PT)])
        pltpu.sync_copy(src_h.at[wid], src_v)
        pltpu.sync_copy(dst_h.at[wid], dst_v)
        plsc.subcore_barrier()

        for b in range(NBUF - 1):
            pltpu.async_copy(y_h.at[src_v.at[b]], rows[b], sems[b])

        def step(j, carry):
            for b in range(NBUF):
                c = j * NBUF + b
                pltpu.make_async_copy(y_h.at[src_v.at[c]],
                                      rows[b], sems[b]).wait()
                cn = jnp.minimum(c + NBUF - 1, chc - 1)
                bn = (b + NBUF - 1) % NBUF
                pltpu.async_copy(y_h.at[src_v.at[cn]], rows[bn], sems[bn])
                pltpu.sync_copy(rows[b], acc.at[dst_v.at[c]], add=True)
            return carry

        lax.fori_loop(0, chc // NBUF, step, 0)
        for k in range(NBUF - 1):
            b = k % NBUF
            pltpu.make_async_copy(y_h.at[src_v.at[0]],
                                  rows[b], sems[b]).wait()
        plsc.subcore_barrier()
        pltpu.sync_copy(acc.at[pl.ds(sid * PT, PT)],
                        out_h.at[cid, pl.ds(sid * PT, PT)])

    return prop_kernel(y, src3, dst3, zeros_blk)


def _edge_layout(edge_index, N, CH_A, CH_B):
    E = edge_index.shape[1]
    cap_a = NS * CH_A * LN
    cap = cap_a + NS * CH_B * LN
    src = jnp.pad(edge_index[0], (0, cap - E))
    dst = jnp.pad(edge_index[1], (0, cap - E), constant_values=N)

    CHM = max(CH_A, CH_B)

    def shape3(a):
        a0 = a[:cap_a].reshape(NS, CH_A, LN)
        a1 = a[cap_a:].reshape(NS, CH_B, LN)
        a0 = jnp.pad(a0, ((0, 0), (0, CHM - CH_A), (0, 0)))
        a1 = jnp.pad(a1, ((0, 0), (0, CHM - CH_B), (0, 0)))
        return jnp.concatenate([a0, a1], axis=0)

    return shape3(src), shape3(dst)


def _deg_inv_sqrt(p0_r, p1_r):
    deg = p0_r[:, 0:1] + p1_r[:, 0:1] + 1.0
    return lax.rsqrt(jnp.maximum(deg, 1e-12))


def _tc_first(xp, W, p0, p1, blk_r):
    NP, K = xp.shape
    F = W.shape[1]
    nb = NP // blk_r

    def body(x_r, w_r, p0_r, p1_r, o_r):
        d = _deg_inv_sqrt(p0_r, p1_r)
        o_r[...] = jnp.dot(x_r[...], w_r[...],
                           preferred_element_type=jnp.float32) * d

    return pl.pallas_call(
        body,
        grid=(nb,),
        in_specs=[
            pl.BlockSpec((blk_r, K), lambda i: (i, 0)),
            pl.BlockSpec((K, F), lambda i: (0, 0)),
            pl.BlockSpec((blk_r, 8), lambda i: (i, 0)),
            pl.BlockSpec((blk_r, 8), lambda i: (i, 0)),
        ],
        out_specs=pl.BlockSpec((blk_r, F), lambda i: (i, 0)),
        out_shape=jax.ShapeDtypeStruct((NP, F), jnp.float32),
    )(xp, W, p0, p1)


def _tc_mid(s0, s1, yp, p0, p1, b2d, W, blk_r):
    NP, K = yp.shape
    F = W.shape[1]
    nb = NP // blk_r

    def body(s0_r, s1_r, y_r, p0_r, p1_r, b_r, w_r, o_r):
        d = _deg_inv_sqrt(p0_r, p1_r)
        z = d * (s0_r[...] + s1_r[...] + y_r[...]) + b_r[...]
        a = jnp.maximum(z, 0.0)
        o_r[...] = jnp.dot(a, w_r[...],
                           preferred_element_type=jnp.float32) * d

    return pl.pallas_call(
        body,
        grid=(nb,),
        in_specs=[
            pl.BlockSpec((blk_r, K), lambda i: (i, 0)),
            pl.BlockSpec((blk_r, K), lambda i: (i, 0)),
            pl.BlockSpec((blk_r, K), lambda i: (i, 0)),
            pl.BlockSpec((blk_r, 8), lambda i: (i, 0)),
            pl.BlockSpec((blk_r, 8), lambda i: (i, 0)),
            pl.BlockSpec((1, K), lambda i: (0, 0)),
            pl.BlockSpec((K, F), lambda i: (0, 0)),
        ],
        out_specs=pl.BlockSpec((blk_r, F), lambda i: (i, 0)),
        out_shape=jax.ShapeDtypeStruct((NP, F), jnp.float32),
    )(s0, s1, yp, p0, p1, b2d, W)


def _tc_final(s0, s1, yp, p0, p1, b2d, blk_r):
    NP, K = yp.shape
    nb = NP // blk_r

    def body(s0_r, s1_r, y_r, p0_r, p1_r, b_r, o_r):
        d = _deg_inv_sqrt(p0_r, p1_r)
        z = d * (s0_r[...] + s1_r[...] + y_r[...]) + b_r[...]
        m = jnp.max(z, axis=1, keepdims=True)
        e = jnp.exp(z - m)
        lse = jnp.log(jnp.sum(e, axis=1, keepdims=True)) + m
        o_r[...] = z - lse

    return pl.pallas_call(
        body,
        grid=(nb,),
        in_specs=[
            pl.BlockSpec((blk_r, K), lambda i: (i, 0)),
            pl.BlockSpec((blk_r, K), lambda i: (i, 0)),
            pl.BlockSpec((blk_r, K), lambda i: (i, 0)),
            pl.BlockSpec((blk_r, 8), lambda i: (i, 0)),
            pl.BlockSpec((blk_r, 8), lambda i: (i, 0)),
            pl.BlockSpec((1, K), lambda i: (0, 0)),
        ],
        out_specs=pl.BlockSpec((blk_r, K), lambda i: (i, 0)),
        out_shape=jax.ShapeDtypeStruct((NP, K), jnp.float32),
    )(s0, s1, yp, p0, p1, b2d)


def kernel(x, edge_index, W1, b1, W2, b2, W3, b3):
    N, K = x.shape
    E = edge_index.shape[1]

    NP = ((N + NS * 8 - 1) // (NS * 8)) * (NS * 8)
    PT = NP // NS

    src1, dst1 = _edge_layout(edge_index, N, 410, 220)
    src2, dst2 = _edge_layout(edge_index, N, 434, 196)
    _, dstd = _edge_layout(edge_index, N, 315, 315)
    xp = jnp.pad(x, ((0, NP - N), (0, 0)))

    z8 = jnp.zeros((PT, 8), jnp.float32)
    ones8 = jnp.ones((LN, 8), jnp.float32)
    zK = jnp.zeros((PT, K), jnp.float32)
    F2 = W2.shape[1]
    zF = jnp.zeros((PT, F2), jnp.float32)

    blk_r = PT

    pdeg = _sc_degree(dstd, z8, ones8, 315, 315)
    p0, p1 = pdeg[0], pdeg[1]

    y1 = _tc_first(xp, W1, p0, p1, blk_r)
    s1 = _sc_propagate(y1, src1, dst1, zK, 5, 410, 220)
    y2 = _tc_mid(s1[0], s1[1], y1, p0, p1, b1.reshape(1, -1), W2, blk_r)
    s2 = _sc_propagate(y2, src2, dst2, zF, 7, 434, 196)
    y3 = _tc_mid(s2[0], s2[1], y2, p0, p1, b2.reshape(1, -1), W3, blk_r)
    s3 = _sc_propagate(y3, src2, dst2, zF, 7, 434, 196)
    out = _tc_final(s3[0], s3[1], y3, p0, p1, b3.reshape(1, -1), blk_r)
    return out[:N]
